# baseline (speedup 1.0000x reference)
import functools

import jax
import jax.numpy as jnp
from jax import lax
from jax.experimental import pallas as pl
from jax.experimental.pallas import tpu as pltpu
from jax.experimental.pallas import tpu_sc as plsc

N = 10000
E = 320000
D = 128
NC = 2
NS = 16
NW = NC * NS
N_PAD = 10240
E_PER_TILE = E // NW
K = 128
NCHUNK = 80
E_PAD = NW * K * NCHUNK
ZROWS = 128
BX = 400

_sc_mesh = plsc.VectorSubcoreMesh(
    core_axis_name="c", subcore_axis_name="s", num_cores=NC, num_subcores=NS
)
_sc_params = pltpu.CompilerParams(needs_layout_passes=False)


def _hist_body(row_hbm, zi_hbm, hist_hbm, idxv, histv):
    cid = lax.axis_index("c")
    sid = lax.axis_index("s")
    wid = cid * NS + sid

    pltpu.sync_copy(zi_hbm, histv)
    pltpu.sync_copy(row_hbm.at[pl.ds(wid * E_PER_TILE, E_PER_TILE)], idxv)
    ones16 = jnp.ones((16,), jnp.int32)

    def ibody(j, cc):
        idx = idxv[pl.ds(j * 16, 16)]
        plsc.addupdate_scatter(histv, [idx], ones16)
        return cc

    lax.fori_loop(0, E_PER_TILE // 16, ibody, 0)
    pltpu.sync_copy(histv, hist_hbm.at[wid])


_hist_call = pl.kernel(
    _hist_body,
    out_type=jax.ShapeDtypeStruct((NW, N_PAD), jnp.int32),
    mesh=_sc_mesh,
    scratch_types=[
        pltpu.VMEM((E_PER_TILE,), jnp.int32),
        pltpu.VMEM((N_PAD,), jnp.int32),
    ],
    compiler_params=_sc_params,
)


def _prep_body(hist_ref, xp_ref, w_ref, xs_ref, dinv_ref):
    h = hist_ref[...].astype(jnp.float32)
    deg = 1.0 + jnp.sum(h, axis=0, keepdims=True)
    dr = lax.rsqrt(deg)
    dc = dr.reshape(1280, 1)
    xw = jnp.dot(xp_ref[...], w_ref[...], preferred_element_type=jnp.float32)
    xs_ref[...] = xw * dc
    dinv_ref[...] = dr.reshape(1, 1, 1280)


_prep_call = pl.pallas_call(
    _prep_body,
    grid=(N_PAD // 1280,),
    in_specs=[
        pl.BlockSpec((NW, 1280), lambda i: (0, i)),
        pl.BlockSpec((1280, D), lambda i: (i, 0)),
        pl.BlockSpec((D, D), lambda i: (0, 0)),
    ],
    out_specs=[
        pl.BlockSpec((1280, D), lambda i: (i, 0)),
        pl.BlockSpec((1, 1, 1280), lambda i: (i, 0, 0)),
    ],
    out_shape=[
        jax.ShapeDtypeStruct((N_PAD, D), jnp.float32),
        jax.ShapeDtypeStruct((N_PAD // 1280, 1, 1280), jnp.float32),
    ],
)


def _scatter_body(xs_hbm, idx_hbm, zeros_hbm, acc0_hbm, acc1_hbm,
                  i0, i1, i2, i3, bufA, bufB, acc_sh,
                  si0, si1, si2, si3, gA, gB):
    cid = lax.axis_index("c")
    sid = lax.axis_index("s")
    wid = cid * NS + sid

    ibufs = [i0, i1, i2, i3]
    isems = [si0, si1, si2, si3]
    dbufs = {0: bufA, 1: bufB}
    gsems = {0: gA, 1: gB}

    rows_per_tile = N_PAD // NS
    pltpu.sync_copy(zeros_hbm, acc_sh.at[pl.ds(sid * rows_per_tile, rows_per_tile)])
    plsc.subcore_barrier()

    def start_idx(c, k4):
        pltpu.async_copy(idx_hbm.at[wid, c], ibufs[k4], isems[k4])

    def wait_idx(k4):
        pltpu.make_async_copy(idx_hbm.at[wid, 0], ibufs[k4], isems[k4]).wait()

    def start_gather(k4, k2):
        wait_idx(k4)
        pltpu.async_copy(xs_hbm.at[ibufs[k4].at[0]], dbufs[k2], gsems[k2])

    def wait_gather(k2):
        pltpu.make_async_copy(xs_hbm.at[ibufs[0].at[0]], dbufs[k2],
                              gsems[k2]).wait()

    def scat(k4, k2):
        pltpu.sync_copy(dbufs[k2], acc_sh.at[ibufs[k4].at[1]], add=True)

    for c in range(4):
        start_idx(c, c)
    start_gather(0, 0)

    def step(c, k4, k2, prefetch, next_gather):
        if next_gather:
            start_gather((k4 + 1) % 4, k2 ^ 1)
        wait_gather(k2)
        scat(k4, k2)
        if prefetch:
            start_idx_c = c + 4
            start_idx(start_idx_c, k4)

    def body(j, carry):
        c0 = 4 * j
        for k in range(4):
            step(c0 + k, k, k % 2, True, True)
        return carry

    lax.fori_loop(0, NCHUNK // 4 - 1, body, 0)
    for k in range(4):
        step(NCHUNK - 4 + k, k, k % 2, False, k < 3)
    plsc.subcore_barrier()

    @pl.when(cid == 0)
    def _():
        pltpu.sync_copy(
            acc_sh.at[pl.ds(sid * rows_per_tile, rows_per_tile)],
            acc0_hbm.at[pl.ds(sid * rows_per_tile, rows_per_tile)],
        )

    @pl.when(cid == 1)
    def _():
        pltpu.sync_copy(
            acc_sh.at[pl.ds(sid * rows_per_tile, rows_per_tile)],
            acc1_hbm.at[pl.ds(sid * rows_per_tile, rows_per_tile)],
        )


_scatter_call = pl.kernel(
    _scatter_body,
    out_type=[
        jax.ShapeDtypeStruct((N_PAD, D), jnp.float32),
        jax.ShapeDtypeStruct((N_PAD, D), jnp.float32),
    ],
    mesh=_sc_mesh,
    scratch_types=(
        [pltpu.VMEM((2, K), jnp.int32)] * 4
        + [pltpu.VMEM((K, D), jnp.float32)] * 2
        + [pltpu.VMEM_SHARED((N_PAD, D), jnp.float32)]
        + [pltpu.SemaphoreType.DMA] * 6
    ),
    compiler_params=_sc_params,
)


def _fin_body(a0_ref, a1_ref, xs_ref, dv_ref, b_ref, o_ref):
    s = (a0_ref[...] + a1_ref[...] + xs_ref[...]) * dv_ref[...] + b_ref[...]
    o_ref[...] = jnp.maximum(s, 0.0)


_fin_call = pl.pallas_call(
    _fin_body,
    grid=(N // BX,),
    in_specs=[
        pl.BlockSpec((BX, D), lambda i: (i, 0)),
        pl.BlockSpec((BX, D), lambda i: (i, 0)),
        pl.BlockSpec((BX, D), lambda i: (i, 0)),
        pl.BlockSpec((BX, 1), lambda i: (i, 0)),
        pl.BlockSpec((1, D), lambda i: (0, 0)),
    ],
    out_specs=pl.BlockSpec((BX, D), lambda i: (i, 0)),
    out_shape=jax.ShapeDtypeStruct((N, D), jnp.float32),
)


def kernel(input_x, input_e, W, b):
    row = input_e[0]
    col = input_e[1]
    hist = _hist_call(row, jnp.zeros((N_PAD,), jnp.int32))
    x_pad = jnp.pad(input_x, ((0, N_PAD - N), (0, 0)))
    xs, dinv3 = _prep_call(hist, x_pad, W)
    dinvp = dinv3.reshape(N_PAD, 1)
    zeros = jnp.zeros((N_PAD // NS, D), jnp.float32)
    npad_e = E_PAD - E
    colp = jnp.concatenate([col, jnp.zeros((npad_e,), jnp.int32)])
    rowp = jnp.concatenate([row, jnp.full((npad_e,), N_PAD - 1, jnp.int32)])
    col3 = colp.reshape(NW, NCHUNK, 1, K)
    row3 = rowp.reshape(NW, NCHUNK, 1, K)
    idx4 = jnp.concatenate([col3, row3], axis=2)
    acc0, acc1 = _scatter_call(xs, idx4, zeros)
    out = _fin_call(acc0, acc1, xs, dinvp, b.reshape(1, D))
    return out


# speedup vs baseline: 2.6004x; 2.6004x over previous
import jax
import jax.numpy as jnp
from jax import lax
from jax.experimental import pallas as pl
from jax.experimental.pallas import tpu as pltpu
from jax.experimental.pallas import tpu_sc as plsc

N = 10000
E = 320000
D = 128
NC = 2
NS = 16
NW = NC * NS
N_PAD = 10240
E_PER_TILE = E // NW
K = 100
NCHUNK = E_PER_TILE // K
BX = 400

_sc_mesh = plsc.VectorSubcoreMesh(
    core_axis_name="c", subcore_axis_name="s", num_cores=NC, num_subcores=NS
)
_sc_params = pltpu.CompilerParams(needs_layout_passes=False)


def _hist_body(row_hbm, zi_hbm, hist_hbm, idxv, histv):
    cid = lax.axis_index("c")
    sid = lax.axis_index("s")
    wid = cid * NS + sid

    pltpu.sync_copy(zi_hbm, histv)
    pltpu.sync_copy(row_hbm.at[pl.ds(wid * E_PER_TILE, E_PER_TILE)], idxv)
    ones16 = jnp.ones((16,), jnp.int32)

    def ibody(j, cc):
        idx = idxv[pl.ds(j * 16, 16)]
        plsc.addupdate_scatter(histv, [idx], ones16)
        return cc

    lax.fori_loop(0, E_PER_TILE // 16, ibody, 0)
    pltpu.sync_copy(histv, hist_hbm.at[wid])


_hist_call = pl.kernel(
    _hist_body,
    out_type=jax.ShapeDtypeStruct((NW, N_PAD), jnp.int32),
    mesh=_sc_mesh,
    scratch_types=[
        pltpu.VMEM((E_PER_TILE,), jnp.int32),
        pltpu.VMEM((N_PAD,), jnp.int32),
    ],
    compiler_params=_sc_params,
)


def _prep_body(hist_ref, xp_ref, w_ref, xs_ref, dinv_ref):
    h = hist_ref[...].astype(jnp.float32)
    deg = 1.0 + jnp.sum(h, axis=0, keepdims=True)
    dr = lax.rsqrt(deg)
    dc = dr.reshape(1280, 1)
    xw = jnp.dot(xp_ref[...], w_ref[...], preferred_element_type=jnp.float32)
    xs_ref[...] = xw * dc
    dinv_ref[...] = dr.reshape(1, 1, 1280)


_prep_call = pl.pallas_call(
    _prep_body,
    grid=(N_PAD // 1280,),
    in_specs=[
        pl.BlockSpec((NW, 1280), lambda i: (0, i)),
        pl.BlockSpec((1280, D), lambda i: (i, 0)),
        pl.BlockSpec((D, D), lambda i: (0, 0)),
    ],
    out_specs=[
        pl.BlockSpec((1280, D), lambda i: (i, 0)),
        pl.BlockSpec((1, 1, 1280), lambda i: (i, 0, 0)),
    ],
    out_shape=[
        jax.ShapeDtypeStruct((N_PAD, D), jnp.float32),
        jax.ShapeDtypeStruct((N_PAD // 1280, 1, 1280), jnp.float32),
    ],
)


def _scatter_body(xs_hbm, idx_hbm, zeros_hbm, acc0_hbm, acc1_hbm,
                  i0, i1, i2, i3, bufA, bufB, acc_sh,
                  si0, si1, si2, si3, gA, gB):
    cid = lax.axis_index("c")
    sid = lax.axis_index("s")
    wid = cid * NS + sid

    ibufs = [i0, i1, i2, i3]
    isems = [si0, si1, si2, si3]
    dbufs = {0: bufA, 1: bufB}
    gsems = {0: gA, 1: gB}

    rows_per_tile = N_PAD // NS
    pltpu.sync_copy(zeros_hbm, acc_sh.at[pl.ds(sid * rows_per_tile, rows_per_tile)])
    plsc.subcore_barrier()

    def start_idx(c, k4):
        pltpu.async_copy(idx_hbm.at[wid, c], ibufs[k4], isems[k4])

    def wait_idx(k4):
        pltpu.make_async_copy(idx_hbm.at[wid, 0], ibufs[k4], isems[k4]).wait()

    def start_gather(k4, k2):
        wait_idx(k4)
        pltpu.async_copy(xs_hbm.at[ibufs[k4].at[0]], dbufs[k2], gsems[k2])

    def wait_gather(k2):
        pltpu.make_async_copy(xs_hbm.at[ibufs[0].at[0]], dbufs[k2],
                              gsems[k2]).wait()

    def scat(k4, k2):
        pltpu.sync_copy(dbufs[k2], acc_sh.at[ibufs[k4].at[1]], add=True)

    for c in range(4):
        start_idx(c, c)
    start_gather(0, 0)

    def step(c, k4, k2, prefetch, next_gather):
        if next_gather:
            start_gather((k4 + 1) % 4, k2 ^ 1)
        wait_gather(k2)
        scat(k4, k2)
        if prefetch:
            start_idx_c = c + 4
            start_idx(start_idx_c, k4)

    def body(j, carry):
        c0 = 4 * j
        for k in range(4):
            step(c0 + k, k, k % 2, True, True)
        return carry

    lax.fori_loop(0, NCHUNK // 4 - 1, body, 0)
    for k in range(4):
        step(NCHUNK - 4 + k, k, k % 2, False, k < 3)
    plsc.subcore_barrier()

    @pl.when(cid == 0)
    def _():
        pltpu.sync_copy(
            acc_sh.at[pl.ds(sid * rows_per_tile, rows_per_tile)],
            acc0_hbm.at[pl.ds(sid * rows_per_tile, rows_per_tile)],
        )

    @pl.when(cid == 1)
    def _():
        pltpu.sync_copy(
            acc_sh.at[pl.ds(sid * rows_per_tile, rows_per_tile)],
            acc1_hbm.at[pl.ds(sid * rows_per_tile, rows_per_tile)],
        )


_scatter_call = pl.kernel(
    _scatter_body,
    out_type=[
        jax.ShapeDtypeStruct((N_PAD, D), jnp.float32),
        jax.ShapeDtypeStruct((N_PAD, D), jnp.float32),
    ],
    mesh=_sc_mesh,
    scratch_types=(
        [pltpu.VMEM((2, K), jnp.int32)] * 4
        + [pltpu.VMEM((K, D), jnp.float32)] * 2
        + [pltpu.VMEM_SHARED((N_PAD, D), jnp.float32)]
        + [pltpu.SemaphoreType.DMA] * 6
    ),
    compiler_params=_sc_params,
)


def _fin_body(a0_ref, a1_ref, xs_ref, dv_ref, b_ref, o_ref):
    s = (a0_ref[...] + a1_ref[...] + xs_ref[...]) * dv_ref[...] + b_ref[...]
    o_ref[...] = jnp.maximum(s, 0.0)


_fin_call = pl.pallas_call(
    _fin_body,
    grid=(N // BX,),
    in_specs=[
        pl.BlockSpec((BX, D), lambda i: (i, 0)),
        pl.BlockSpec((BX, D), lambda i: (i, 0)),
        pl.BlockSpec((BX, D), lambda i: (i, 0)),
        pl.BlockSpec((BX, 1), lambda i: (i, 0)),
        pl.BlockSpec((1, D), lambda i: (0, 0)),
    ],
    out_specs=pl.BlockSpec((BX, D), lambda i: (i, 0)),
    out_shape=jax.ShapeDtypeStruct((N, D), jnp.float32),
)


def kernel(input_x, input_e, W, b):
    row = input_e[0]
    col = input_e[1]
    hist = _hist_call(row, jnp.zeros((N_PAD,), jnp.int32))
    x_pad = jnp.pad(input_x, ((0, N_PAD - N), (0, 0)))
    xs, dinv3 = _prep_call(hist, x_pad, W)
    dinvp = dinv3.reshape(N_PAD, 1)
    zeros = jnp.zeros((N_PAD // NS, D), jnp.float32)
    col3 = col.reshape(NW, NCHUNK, 1, K)
    row3 = row.reshape(NW, NCHUNK, 1, K)
    idx4 = jnp.concatenate([col3, row3], axis=2)
    acc0, acc1 = _scatter_call(xs, idx4, zeros)
    out = _fin_call(acc0, acc1, xs, dinvp, b.reshape(1, D))
    return out
